# baseline (speedup 1.0000x reference)
import functools

import jax
import jax.numpy as jnp
from jax import lax
from jax.experimental import pallas as pl
from jax.experimental.pallas import tpu as pltpu
from jax.experimental.pallas import tpu_sc as plsc

N = 100000
C = 9
V = 200
D = 512
W = D // 2
B = 8
G = B * C
NBLK = N // B
NW = 32


@functools.partial(
    pl.kernel,
    out_type=jax.ShapeDtypeStruct((2 * N, W), jnp.bfloat16),
    mesh=plsc.VectorSubcoreMesh(core_axis_name="c", subcore_axis_name="s"),
    compiler_params=pltpu.CompilerParams(use_tc_tiling_on_sc=True),
    scratch_types=[
        pltpu.VMEM((2 * G,), jnp.int32),
        pltpu.VMEM((2, 2 * G, W), jnp.bfloat16),
        pltpu.VMEM((2 * B, W), jnp.bfloat16),
        pltpu.SemaphoreType.DMA((2,)),
        pltpu.SemaphoreType.DMA((2,)),
    ],
)
def _atom_encoder_sc(idx_hbm, tabs_hbm, out_hbm, idx_v, rows_v, out_v,
                     sem_idx, sem_g):
    w = lax.axis_index("s") * 2 + lax.axis_index("c")
    nblk_w = jnp.where(w < 20, 391, 390)
    blk0 = w * 390 + jnp.minimum(w, 20)

    def idx_copy(blk, slot, sem):
        return pltpu.make_async_copy(
            idx_hbm.at[pl.ds(blk * G, G)],
            idx_v.at[pl.ds(pl.multiple_of(slot * G, 8), G)], sem)

    def gather(slot, sem):
        return pltpu.make_async_copy(
            tabs_hbm.at[idx_v.at[pl.ds(pl.multiple_of(slot * G, 8), G)]],
            rows_v.at[slot].bitcast(jnp.int32), sem)

    idx_copy(blk0, 0, sem_idx.at[0]).start()
    idx_copy(blk0, 0, sem_idx.at[0]).wait()
    gather(0, sem_g.at[0]).start()

    @pl.when(nblk_w > 1)
    def _():
        idx_copy(blk0 + 1, 1, sem_idx.at[1]).start()

    def block_step(k, carry):
        blk = blk0 + k
        buf = lax.rem(k, 2)
        nbuf = 1 - buf

        gather(buf, sem_g.at[buf]).wait()

        @pl.when(k + 2 < nblk_w)
        def _():
            idx_copy(blk + 2, buf, sem_idx.at[buf]).start()

        @pl.when(k + 1 < nblk_w)
        def _():
            idx_copy(blk + 1, nbuf, sem_idx.at[nbuf]).wait()
            gather(nbuf, sem_g.at[nbuf]).start()

        def row_step(n, c1):
            def col_step(c, c2):
                sl = pl.ds(c * 16, 16)
                r0 = pl.multiple_of(n * 2 * C, 2)
                acc = rows_v[buf, pl.ds(r0, 2), sl]
                for i in range(1, C):
                    acc = acc + rows_v[buf, pl.ds(r0 + 2 * i, 2), sl]
                out_v[pl.ds(pl.multiple_of(n * 2, 2), 2), sl] = acc
                return c2

            return lax.fori_loop(0, W // 16, col_step, c1)

        lax.fori_loop(0, B, row_step, 0)
        pltpu.sync_copy(out_v, out_hbm.at[pl.ds(blk * 2 * B, 2 * B)])
        return carry

    lax.fori_loop(0, nblk_w, block_step, 0)


def kernel(x, tables):
    offs = (jnp.arange(C, dtype=jnp.int32) * V)[None, :]
    idx = (x.astype(jnp.int32) + offs).reshape(N * C)
    tabs16 = tables.astype(jnp.bfloat16).reshape(C * V, W, 2)
    tabs = lax.bitcast_convert_type(tabs16, jnp.int32)
    split = _atom_encoder_sc(idx, tabs)
    out16 = split.reshape(N, 2, W).transpose(0, 2, 1).reshape(N, D)
    return out16.astype(jnp.float32)
